# Initial kernel scaffold; baseline (speedup 1.0000x reference)
import jax
import jax.numpy as jnp
from jax.experimental import pallas as pl


def kernel(x, edge_index, W1, b1, W2, b2):
    raise NotImplementedError("write your pallas kernel here")


# speedup vs baseline: 21.4560x; 21.4560x over previous
import functools

import jax
import jax.numpy as jnp
from jax import lax
from jax.experimental import pallas as pl
from jax.experimental.pallas import tpu as pltpu
from jax.experimental.pallas import tpu_sc as plsc

N = 10000
E = 320000
D = 128
C = 40
CP = 48

NC, NS = 2, 16
NW = NC * NS
CH = 125
NCH_T = E // CH
NCH_W = NCH_T // NW
NP = 10240
RPS = NP // NS

_mesh = plsc.VectorSubcoreMesh(core_axis_name="c", subcore_axis_name="s",
                               num_cores=NC, num_subcores=NS)


@functools.partial(
    pl.kernel,
    out_type=jax.ShapeDtypeStruct((NC, NP), jnp.float32),
    mesh=_mesh,
    scratch_types=[
        pltpu.VMEM((NCH_W, CH), jnp.int32),
        pltpu.VMEM((CH,), jnp.float32),
        pltpu.VMEM_SHARED((NP,), jnp.float32),
    ],
)
def _sc_degree(col_hbm, ones_hbm, zeros_hbm, out_hbm, cidx_v, ones_v, hist_sh):
    c = lax.axis_index("c")
    s = lax.axis_index("s")
    wid = s * NC + c
    pltpu.sync_copy(col_hbm.at[pl.ds(wid * NCH_W, NCH_W)], cidx_v)
    pltpu.sync_copy(ones_hbm.at[pl.ds(0, CH)], ones_v)
    pltpu.sync_copy(zeros_hbm.at[pl.ds(s * RPS, RPS)],
                    hist_sh.at[pl.ds(s * RPS, RPS)])
    plsc.subcore_barrier()

    def step(j, carry):
        pltpu.sync_copy(ones_v, hist_sh.at[cidx_v.at[j]], add=True)
        return carry

    lax.fori_loop(0, NCH_W, step, 0)
    plsc.subcore_barrier()
    pltpu.sync_copy(hist_sh.at[pl.ds(s * RPS, RPS)],
                    out_hbm.at[c, pl.ds(s * RPS, RPS)])


def _make_sc_propagate(width):
    @functools.partial(
        pl.kernel,
        out_type=jax.ShapeDtypeStruct((NC, NP, width), jnp.float32),
        mesh=_mesh,
        scratch_types=[
            pltpu.VMEM((NCH_W, CH), jnp.int32),
            pltpu.VMEM((NCH_W, CH), jnp.int32),
            pltpu.VMEM((CH, width), jnp.float32),
            pltpu.VMEM_SHARED((NP, width), jnp.float32),
        ],
    )
    def _sc_prop(z_hbm, row_hbm, col_hbm, zeros_hbm, out_hbm,
                 ridx_v, cidx_v, rows_v, acc_sh):
        c = lax.axis_index("c")
        s = lax.axis_index("s")
        wid = s * NC + c
        pltpu.sync_copy(row_hbm.at[pl.ds(wid * NCH_W, NCH_W)], ridx_v)
        pltpu.sync_copy(col_hbm.at[pl.ds(wid * NCH_W, NCH_W)], cidx_v)
        pltpu.sync_copy(zeros_hbm.at[pl.ds(s * RPS, RPS)],
                        acc_sh.at[pl.ds(s * RPS, RPS)])
        plsc.subcore_barrier()

        def step(j, carry):
            pltpu.sync_copy(z_hbm.at[ridx_v.at[j]], rows_v)
            pltpu.sync_copy(rows_v, acc_sh.at[cidx_v.at[j]], add=True)
            return carry

        lax.fori_loop(0, NCH_W, step, 0)
        plsc.subcore_barrier()
        pltpu.sync_copy(acc_sh.at[pl.ds(s * RPS, RPS)],
                        out_hbm.at[c, pl.ds(s * RPS, RPS)])

    return _sc_prop


_sc_prop_d = _make_sc_propagate(D)


R = 2000


def _tc1_body(h0, h1, x, w1, z, dis):
    dis_v = lax.rsqrt(h0[...] + h1[...] + 1.0)
    dis[...] = dis_v
    z[...] = dis_v * jnp.dot(x[...], w1[...],
                             preferred_element_type=jnp.float32)


_tc1 = pl.pallas_call(
    _tc1_body,
    grid=(N // R,),
    in_specs=[
        pl.BlockSpec((R, 1), lambda i: (i, 0)),
        pl.BlockSpec((R, 1), lambda i: (i, 0)),
        pl.BlockSpec((R, D), lambda i: (i, 0)),
        pl.BlockSpec((D, D), lambda i: (0, 0)),
    ],
    out_specs=[
        pl.BlockSpec((R, D), lambda i: (i, 0)),
        pl.BlockSpec((R, 1), lambda i: (i, 0)),
    ],
    out_shape=[
        jax.ShapeDtypeStruct((N, D), jnp.float32),
        jax.ShapeDtypeStruct((N, 1), jnp.float32),
    ],
)


def _tc2_body(p0, p1, z1, dis, b1, z2):
    h = jnp.maximum(dis[...] * (p0[...] + p1[...] + z1[...]) + b1[...], 0.0)
    z2[...] = dis[...] * h


_tc2 = pl.pallas_call(
    _tc2_body,
    grid=(N // R,),
    in_specs=[
        pl.BlockSpec((R, D), lambda i: (i, 0)),
        pl.BlockSpec((R, D), lambda i: (i, 0)),
        pl.BlockSpec((R, D), lambda i: (i, 0)),
        pl.BlockSpec((R, 1), lambda i: (i, 0)),
        pl.BlockSpec((1, D), lambda i: (0, 0)),
    ],
    out_specs=pl.BlockSpec((R, D), lambda i: (i, 0)),
    out_shape=jax.ShapeDtypeStruct((N, D), jnp.float32),
)


def _tc3_body(p0, p1, z2, dis, w2, b2, out):
    g = dis[...] * (p0[...] + p1[...] + z2[...])
    o = jnp.dot(g, w2[...], preferred_element_type=jnp.float32) + b2[...]
    col = lax.broadcasted_iota(jnp.int32, o.shape, 1)
    valid = col < C
    om = jnp.where(valid, o, jnp.float32(-1e30))
    mx = jnp.max(om, axis=1, keepdims=True)
    e = jnp.where(valid, jnp.exp(o - mx), 0.0)
    ssum = jnp.sum(e, axis=1, keepdims=True)
    out[...] = (o - mx) - jnp.log(ssum)


_tc3 = pl.pallas_call(
    _tc3_body,
    grid=(N // R,),
    in_specs=[
        pl.BlockSpec((R, D), lambda i: (i, 0)),
        pl.BlockSpec((R, D), lambda i: (i, 0)),
        pl.BlockSpec((R, D), lambda i: (i, 0)),
        pl.BlockSpec((R, 1), lambda i: (i, 0)),
        pl.BlockSpec((D, CP), lambda i: (0, 0)),
        pl.BlockSpec((1, CP), lambda i: (0, 0)),
    ],
    out_specs=pl.BlockSpec((R, CP), lambda i: (i, 0)),
    out_shape=jax.ShapeDtypeStruct((N, CP), jnp.float32),
)


def kernel(x, edge_index, W1, b1, W2, b2):
    row2d = edge_index[0].astype(jnp.int32).reshape(NCH_T, CH)
    col2d = edge_index[1].astype(jnp.int32).reshape(NCH_T, CH)

    zeros1 = jnp.zeros((NP,), jnp.float32)
    ones1 = jnp.ones((128,), jnp.float32)
    zeros_d = jnp.zeros((NP, D), jnp.float32)
    w2p = jnp.concatenate([W2, jnp.zeros((D, CP - C), jnp.float32)], axis=1)
    b1r = b1[None, :]
    b2p = jnp.concatenate([b2, jnp.zeros((CP - C,), jnp.float32)])[None, :]

    hist = _sc_degree(col2d, ones1, zeros1)
    h0 = hist[0, :N, None]
    h1 = hist[1, :N, None]
    z1, dis = _tc1(h0, h1, x, W1)
    p1 = _sc_prop_d(z1, row2d, col2d, zeros_d)
    z2 = _tc2(p1[0, :N], p1[1, :N], z1, dis, b1r)
    p2 = _sc_prop_d(z2, row2d, col2d, zeros_d)
    o = _tc3(p2[0, :N], p2[1, :N], z2, dis, w2p, b2p)
    return o[:, :C]


# speedup vs baseline: 30.4893x; 1.4210x over previous
import functools

import jax
import jax.numpy as jnp
from jax import lax
from jax.experimental import pallas as pl
from jax.experimental.pallas import tpu as pltpu
from jax.experimental.pallas import tpu_sc as plsc

N = 10000
E = 320000
D = 128
C = 40
CP = 48

NC, NS = 2, 16
NW = NC * NS
CH = 125
NCH_T = E // CH
NCH_W = NCH_T // NW
NP = 10240
RPS = NP // NS
G = 16
NG = NCH_W // G

_mesh = plsc.VectorSubcoreMesh(core_axis_name="c", subcore_axis_name="s",
                               num_cores=NC, num_subcores=NS)


@functools.partial(
    pl.kernel,
    out_type=jax.ShapeDtypeStruct((NC, NP), jnp.float32),
    mesh=_mesh,
    scratch_types=[
        pltpu.VMEM((NCH_W, CH), jnp.int32),
        pltpu.VMEM((CH,), jnp.float32),
        pltpu.VMEM_SHARED((NP,), jnp.float32),
    ],
)
def _sc_degree(col_hbm, ones_hbm, zeros_hbm, out_hbm, cidx_v, ones_v, hist_sh):
    c = lax.axis_index("c")
    s = lax.axis_index("s")
    wid = s * NC + c
    pltpu.sync_copy(col_hbm.at[pl.ds(wid * NCH_W, NCH_W)], cidx_v)
    pltpu.sync_copy(ones_hbm.at[pl.ds(0, CH)], ones_v)
    pltpu.sync_copy(zeros_hbm.at[pl.ds(s * RPS, RPS)],
                    hist_sh.at[pl.ds(s * RPS, RPS)])
    plsc.subcore_barrier()

    def step(j, carry):
        pltpu.sync_copy(ones_v, hist_sh.at[cidx_v.at[j]], add=True)
        return carry

    lax.fori_loop(0, NCH_W, step, 0)
    plsc.subcore_barrier()
    pltpu.sync_copy(hist_sh.at[pl.ds(s * RPS, RPS)],
                    out_hbm.at[c, pl.ds(s * RPS, RPS)])


def _make_sc_propagate(width):
    @functools.partial(
        pl.kernel,
        out_type=jax.ShapeDtypeStruct((NC, NP, width), jnp.float32),
        mesh=_mesh,
        scratch_types=[
            pltpu.VMEM((2, G, CH), jnp.int32),
            pltpu.VMEM((2, G, CH), jnp.int32),
            pltpu.VMEM((CH, width), jnp.float32),
            pltpu.VMEM((CH, width), jnp.float32),
            pltpu.VMEM_SHARED((NP, width), jnp.float32),
            pltpu.SemaphoreType.DMA,
            pltpu.SemaphoreType.DMA,
            pltpu.SemaphoreType.DMA,
        ],
    )
    def _sc_prop(z_hbm, row_hbm, col_hbm, zeros_hbm, out_hbm,
                 ridx_v, cidx_v, rows0_v, rows1_v, acc_sh, sem0, sem1, isem):
        c = lax.axis_index("c")
        s = lax.axis_index("s")
        wid = s * NC + c
        base = wid * NCH_W
        pltpu.sync_copy(row_hbm.at[pl.ds(base, G)], ridx_v.at[0])
        pltpu.sync_copy(col_hbm.at[pl.ds(base, G)], cidx_v.at[0])
        pltpu.sync_copy(zeros_hbm.at[pl.ds(s * RPS, RPS)],
                        acc_sh.at[pl.ds(s * RPS, RPS)])
        plsc.subcore_barrier()

        for g in range(NG):
            slot, nxt = g % 2, (g + 1) % 2
            if g + 1 < NG:
                gb = base + (g + 1) * G
                pltpu.async_copy(row_hbm.at[pl.ds(gb, G)], ridx_v.at[nxt], isem)
                pltpu.async_copy(col_hbm.at[pl.ds(gb, G)], cidx_v.at[nxt], isem)
            pltpu.async_copy(z_hbm.at[ridx_v.at[slot, 0]], rows0_v, sem0)

            def step(i, carry, slot=slot):
                j = 2 * i
                cp1 = pltpu.async_copy(
                    z_hbm.at[ridx_v.at[slot, j + 1]], rows1_v, sem1)
                pltpu.make_async_copy(
                    z_hbm.at[ridx_v.at[slot, j]], rows0_v, sem0).wait()
                pltpu.sync_copy(rows0_v, acc_sh.at[cidx_v.at[slot, j]],
                                add=True)

                @pl.when(j + 2 < G)
                def _prefetch():
                    pltpu.async_copy(
                        z_hbm.at[ridx_v.at[slot, j + 2]], rows0_v, sem0)

                cp1.wait()
                pltpu.sync_copy(rows1_v, acc_sh.at[cidx_v.at[slot, j + 1]],
                                add=True)
                return carry

            lax.fori_loop(0, G // 2, step, 0)
            if g + 1 < NG:
                gb = base + (g + 1) * G
                pltpu.make_async_copy(
                    row_hbm.at[pl.ds(gb, G)], ridx_v.at[nxt], isem).wait()
                pltpu.make_async_copy(
                    col_hbm.at[pl.ds(gb, G)], cidx_v.at[nxt], isem).wait()
        plsc.subcore_barrier()
        pltpu.sync_copy(acc_sh.at[pl.ds(s * RPS, RPS)],
                        out_hbm.at[c, pl.ds(s * RPS, RPS)])

    return _sc_prop


_sc_prop_d = _make_sc_propagate(D)


R = 2000


def _tc1_body(h0, h1, x, w1, z, dis):
    dis_v = lax.rsqrt(h0[...] + h1[...] + 1.0)
    dis[...] = dis_v
    z[...] = dis_v * jnp.dot(x[...], w1[...],
                             preferred_element_type=jnp.float32)


_tc1 = pl.pallas_call(
    _tc1_body,
    grid=(N // R,),
    in_specs=[
        pl.BlockSpec((R, 1), lambda i: (i, 0)),
        pl.BlockSpec((R, 1), lambda i: (i, 0)),
        pl.BlockSpec((R, D), lambda i: (i, 0)),
        pl.BlockSpec((D, D), lambda i: (0, 0)),
    ],
    out_specs=[
        pl.BlockSpec((R, D), lambda i: (i, 0)),
        pl.BlockSpec((R, 1), lambda i: (i, 0)),
    ],
    out_shape=[
        jax.ShapeDtypeStruct((N, D), jnp.float32),
        jax.ShapeDtypeStruct((N, 1), jnp.float32),
    ],
)


def _tc2_body(p, z1, dis, b1, z2):
    h = jnp.maximum(dis[...] * (p[0] + p[1] + z1[...]) + b1[...], 0.0)
    z2[...] = dis[...] * h


_tc2 = pl.pallas_call(
    _tc2_body,
    grid=(N // R,),
    in_specs=[
        pl.BlockSpec((2, R, D), lambda i: (0, i, 0)),
        pl.BlockSpec((R, D), lambda i: (i, 0)),
        pl.BlockSpec((R, 1), lambda i: (i, 0)),
        pl.BlockSpec((1, D), lambda i: (0, 0)),
    ],
    out_specs=pl.BlockSpec((R, D), lambda i: (i, 0)),
    out_shape=jax.ShapeDtypeStruct((N, D), jnp.float32),
)


def _tc3_body(p, z2, dis, w2, b2, out):
    g = dis[...] * (p[0] + p[1] + z2[...])
    o = jnp.dot(g, w2[...], preferred_element_type=jnp.float32) + b2[...]
    col = lax.broadcasted_iota(jnp.int32, o.shape, 1)
    valid = col < C
    om = jnp.where(valid, o, jnp.float32(-1e30))
    mx = jnp.max(om, axis=1, keepdims=True)
    e = jnp.where(valid, jnp.exp(o - mx), 0.0)
    ssum = jnp.sum(e, axis=1, keepdims=True)
    out[...] = (o - mx) - jnp.log(ssum)


_tc3 = pl.pallas_call(
    _tc3_body,
    grid=(N // R,),
    in_specs=[
        pl.BlockSpec((2, R, D), lambda i: (0, i, 0)),
        pl.BlockSpec((R, D), lambda i: (i, 0)),
        pl.BlockSpec((R, 1), lambda i: (i, 0)),
        pl.BlockSpec((D, CP), lambda i: (0, 0)),
        pl.BlockSpec((1, CP), lambda i: (0, 0)),
    ],
    out_specs=pl.BlockSpec((R, CP), lambda i: (i, 0)),
    out_shape=jax.ShapeDtypeStruct((N, CP), jnp.float32),
)


def kernel(x, edge_index, W1, b1, W2, b2):
    row2d = edge_index[0].astype(jnp.int32).reshape(NCH_T, CH)
    col2d = edge_index[1].astype(jnp.int32).reshape(NCH_T, CH)

    zeros1 = jnp.zeros((NP,), jnp.float32)
    ones1 = jnp.ones((128,), jnp.float32)
    zeros_d = jnp.zeros((NP, D), jnp.float32)
    w2p = jnp.concatenate([W2, jnp.zeros((D, CP - C), jnp.float32)], axis=1)
    b1r = b1[None, :]
    b2p = jnp.concatenate([b2, jnp.zeros((CP - C,), jnp.float32)])[None, :]

    hist = _sc_degree(col2d, ones1, zeros1)
    h0 = hist[0, :N, None]
    h1 = hist[1, :N, None]
    z1, dis = _tc1(h0, h1, x, W1)
    p1 = _sc_prop_d(z1, row2d, col2d, zeros_d)
    z2 = _tc2(p1, z1, dis, b1r)
    p2 = _sc_prop_d(z2, row2d, col2d, zeros_d)
    o = _tc3(p2, z2, dis, w2p, b2p)
    return o[:, :C]
